# Initial kernel scaffold; baseline (speedup 1.0000x reference)
import jax
import jax.numpy as jnp
from jax.experimental import pallas as pl


def kernel(x, w0, b0, w1, b1, w2, b2):
    raise NotImplementedError("write your pallas kernel here")


# speedup vs baseline: 2.2550x; 2.2550x over previous
import functools

import jax
import jax.numpy as jnp
from jax.experimental import pallas as pl
from jax.experimental.pallas import tpu as pltpu


def _sigmoid(h):
    return 1.0 / (1.0 + jnp.exp(-h))


def _st_kernel(x_ref, w0_ref, b0_ref, w1_ref, b1_ref, w2_ref, b2_ref,
               o_ref, xpad_ref, *, kernel_size, pad):
    xv = x_ref[0]
    S, N = xv.shape

    xpad_ref[pl.ds(pad, S), :] = xv
    xpad_ref[pl.ds(0, pad), :] = jnp.broadcast_to(xv[0:1, :], (pad, N))
    xpad_ref[pl.ds(pad + S, pad), :] = jnp.broadcast_to(xv[S - 1:S, :], (pad, N))

    acc = jnp.zeros((S, N), jnp.float32)
    for i in range(kernel_size):
        acc = acc + xpad_ref[pl.ds(i, S), :]
    mean = acc * (1.0 / kernel_size)
    res = xv - mean

    v = jnp.concatenate([res, mean], axis=1).astype(jnp.bfloat16)

    h = jnp.dot(w0_ref[...], v, preferred_element_type=jnp.float32)
    h = _sigmoid(h + b0_ref[...]).astype(jnp.bfloat16)
    h = jnp.dot(w1_ref[...], h, preferred_element_type=jnp.float32)
    h = _sigmoid(h + b1_ref[...]).astype(jnp.bfloat16)
    h = jnp.dot(w2_ref[...], h, preferred_element_type=jnp.float32) + b2_ref[...]

    o_ref[0] = (_sigmoid(h[:, :N]) + _sigmoid(h[:, N:])).astype(o_ref.dtype)


def kernel(x, w0, b0, w1, b1, w2, b2):
    B, S, N = x.shape
    D = w0.shape[0]
    kernel_size = 25
    pad = (kernel_size - 1) // 2

    w0b = w0.astype(jnp.bfloat16)
    w1b = w1.astype(jnp.bfloat16)
    w2b = w2.astype(jnp.bfloat16)

    body = functools.partial(_st_kernel, kernel_size=kernel_size, pad=pad)
    out = pl.pallas_call(
        body,
        out_shape=jax.ShapeDtypeStruct((B, D, N), x.dtype),
        grid_spec=pltpu.PrefetchScalarGridSpec(
            num_scalar_prefetch=0,
            grid=(B,),
            in_specs=[
                pl.BlockSpec((1, S, N), lambda j: (j, 0, 0)),
                pl.BlockSpec((D, S), lambda j: (0, 0)),
                pl.BlockSpec((D, 1), lambda j: (0, 0)),
                pl.BlockSpec((D, D), lambda j: (0, 0)),
                pl.BlockSpec((D, 1), lambda j: (0, 0)),
                pl.BlockSpec((D, D), lambda j: (0, 0)),
                pl.BlockSpec((D, 1), lambda j: (0, 0)),
            ],
            out_specs=pl.BlockSpec((1, D, N), lambda j: (j, 0, 0)),
            scratch_shapes=[pltpu.VMEM((S + 2 * pad, N), jnp.float32)],
        ),
        compiler_params=pltpu.CompilerParams(
            dimension_semantics=("parallel",)),
    )(x, w0b, b0.reshape(D, 1), w1b, b1.reshape(D, 1), w2b, b2.reshape(D, 1))
    return out


# speedup vs baseline: 4.0147x; 1.7804x over previous
import functools

import numpy as np

import jax
import jax.numpy as jnp
from jax.experimental import pallas as pl
from jax.experimental.pallas import tpu as pltpu


def _sigmoid(h):
    return 1.0 / (1.0 + jnp.exp(-h))


def _avg_matrix(S, kernel_size):
    pad = (kernel_size - 1) // 2
    A = np.zeros((S, S), np.float32)
    for i in range(S):
        for t in range(kernel_size):
            j = min(max(i + t - pad, 0), S - 1)
            A[i, j] += 1.0 / kernel_size
    return A


def _prep_kernel(w0_ref, a_ref, wr_ref, wm_ref):
    wm = jnp.dot(w0_ref[...], a_ref[...], preferred_element_type=jnp.float32)
    wm_ref[...] = wm.astype(jnp.bfloat16)
    wr_ref[...] = (w0_ref[...] - wm).astype(jnp.bfloat16)


def _main_kernel(x_ref, wr_ref, wm_ref, b0_ref, w1_ref, b1_ref, w2_ref,
                 b2_ref, o_ref):
    G, S, N = x_ref.shape
    GN = G * N
    xc = jnp.concatenate([x_ref[g] for g in range(G)],
                         axis=1).astype(jnp.bfloat16)

    b0 = b0_ref[...]
    ht = _sigmoid(jnp.dot(wr_ref[...], xc,
                          preferred_element_type=jnp.float32) + b0)
    hr = _sigmoid(jnp.dot(wm_ref[...], xc,
                          preferred_element_type=jnp.float32) + b0)
    v = jnp.concatenate([ht, hr], axis=1).astype(jnp.bfloat16)
    h = _sigmoid(jnp.dot(w1_ref[...], v,
                         preferred_element_type=jnp.float32) + b1_ref[...])
    h = h.astype(jnp.bfloat16)
    h = jnp.dot(w2_ref[...], h,
                preferred_element_type=jnp.float32) + b2_ref[...]
    for g in range(G):
        o_ref[g] = (_sigmoid(h[:, g * N:(g + 1) * N]) +
                    _sigmoid(h[:, GN + g * N:GN + (g + 1) * N])
                    ).astype(o_ref.dtype)


def kernel(x, w0, b0, w1, b1, w2, b2):
    B, S, N = x.shape
    D = w0.shape[0]
    kernel_size = 25

    A = jnp.asarray(_avg_matrix(S, kernel_size))
    wr, wm = pl.pallas_call(
        _prep_kernel,
        out_shape=(
            jax.ShapeDtypeStruct((D, S), jnp.bfloat16),
            jax.ShapeDtypeStruct((D, S), jnp.bfloat16),
        ),
    )(w0, A)

    G = 4 if B % 4 == 0 else 1
    out = pl.pallas_call(
        _main_kernel,
        out_shape=jax.ShapeDtypeStruct((B, D, N), x.dtype),
        grid_spec=pltpu.PrefetchScalarGridSpec(
            num_scalar_prefetch=0,
            grid=(B // G,),
            in_specs=[
                pl.BlockSpec((G, S, N), lambda j: (j, 0, 0)),
                pl.BlockSpec((D, S), lambda j: (0, 0)),
                pl.BlockSpec((D, S), lambda j: (0, 0)),
                pl.BlockSpec((D, 1), lambda j: (0, 0)),
                pl.BlockSpec((D, D), lambda j: (0, 0)),
                pl.BlockSpec((D, 1), lambda j: (0, 0)),
                pl.BlockSpec((D, D), lambda j: (0, 0)),
                pl.BlockSpec((D, 1), lambda j: (0, 0)),
            ],
            out_specs=pl.BlockSpec((G, D, N), lambda j: (j, 0, 0)),
        ),
        compiler_params=pltpu.CompilerParams(
            dimension_semantics=("parallel",)),
    )(x, wr, wm, b0.reshape(D, 1), w1.astype(jnp.bfloat16),
      b1.reshape(D, 1), w2.astype(jnp.bfloat16), b2.reshape(D, 1))
    return out


# speedup vs baseline: 4.7657x; 1.1871x over previous
import functools

import numpy as np

import jax
import jax.numpy as jnp
from jax.experimental import pallas as pl
from jax.experimental.pallas import tpu as pltpu


def _avg_matrix(S, kernel_size):
    pad = (kernel_size - 1) // 2
    A = np.zeros((S, S), np.float32)
    for i in range(S):
        for t in range(kernel_size):
            j = min(max(i + t - pad, 0), S - 1)
            A[i, j] += 1.0 / kernel_size
    return A


def _prep_kernel(w0_ref, a_ref, b0_ref, w1_ref, b1_ref, w2_ref, b2_ref,
                 wr_ref, wm_ref, b0o_ref, w1o_ref, b1o_ref, w2o_ref, b2o_ref):
    w0 = w0_ref[...]
    wm = jnp.dot(w0, a_ref[...], preferred_element_type=jnp.float32)
    wm_ref[...] = (0.5 * wm).astype(jnp.bfloat16)
    wr_ref[...] = (0.5 * (w0 - wm)).astype(jnp.bfloat16)
    b0o_ref[...] = 0.5 * b0_ref[...]
    w1 = w1_ref[...]
    w1o_ref[...] = (0.25 * w1).astype(jnp.bfloat16)
    b1o_ref[...] = 0.5 * b1_ref[...] + 0.25 * jnp.sum(w1, axis=1, keepdims=True)
    w2 = w2_ref[...]
    w2o_ref[...] = (0.25 * w2).astype(jnp.bfloat16)
    b2o_ref[...] = 0.5 * b2_ref[...] + 0.25 * jnp.sum(w2, axis=1, keepdims=True)


def _main_kernel(x_ref, wr_ref, wm_ref, b0_ref, w1_ref, b1_ref, w2_ref,
                 b2_ref, o_ref):
    G, S, N = x_ref.shape
    GN = G * N
    xc = jnp.concatenate([x_ref[g] for g in range(G)],
                         axis=1).astype(jnp.bfloat16)

    b0 = b0_ref[...]
    ut = jnp.tanh(jnp.dot(wr_ref[...], xc,
                          preferred_element_type=jnp.float32) + b0)
    ur = jnp.tanh(jnp.dot(wm_ref[...], xc,
                          preferred_element_type=jnp.float32) + b0)
    v = jnp.concatenate([ut, ur], axis=1).astype(jnp.bfloat16)
    u = jnp.tanh(jnp.dot(w1_ref[...], v,
                         preferred_element_type=jnp.float32) + b1_ref[...])
    u = u.astype(jnp.bfloat16)
    h = jnp.tanh(jnp.dot(w2_ref[...], u,
                         preferred_element_type=jnp.float32) + b2_ref[...])
    for g in range(G):
        o_ref[g] = (1.0 + 0.5 * (h[:, g * N:(g + 1) * N] +
                                 h[:, GN + g * N:GN + (g + 1) * N])
                    ).astype(o_ref.dtype)


def kernel(x, w0, b0, w1, b1, w2, b2):
    B, S, N = x.shape
    D = w0.shape[0]
    kernel_size = 25

    A = jnp.asarray(_avg_matrix(S, kernel_size))
    wr, wm, b0p, w1p, b1p, w2p, b2p = pl.pallas_call(
        _prep_kernel,
        out_shape=(
            jax.ShapeDtypeStruct((D, S), jnp.bfloat16),
            jax.ShapeDtypeStruct((D, S), jnp.bfloat16),
            jax.ShapeDtypeStruct((D, 1), jnp.float32),
            jax.ShapeDtypeStruct((D, D), jnp.bfloat16),
            jax.ShapeDtypeStruct((D, 1), jnp.float32),
            jax.ShapeDtypeStruct((D, D), jnp.bfloat16),
            jax.ShapeDtypeStruct((D, 1), jnp.float32),
        ),
    )(w0, A, b0.reshape(D, 1), w1, b1.reshape(D, 1), w2, b2.reshape(D, 1))

    G = 4 if B % 4 == 0 else 1
    out = pl.pallas_call(
        _main_kernel,
        out_shape=jax.ShapeDtypeStruct((B, D, N), x.dtype),
        grid_spec=pltpu.PrefetchScalarGridSpec(
            num_scalar_prefetch=0,
            grid=(B // G,),
            in_specs=[
                pl.BlockSpec((G, S, N), lambda j: (j, 0, 0)),
                pl.BlockSpec((D, S), lambda j: (0, 0)),
                pl.BlockSpec((D, S), lambda j: (0, 0)),
                pl.BlockSpec((D, 1), lambda j: (0, 0)),
                pl.BlockSpec((D, D), lambda j: (0, 0)),
                pl.BlockSpec((D, 1), lambda j: (0, 0)),
                pl.BlockSpec((D, D), lambda j: (0, 0)),
                pl.BlockSpec((D, 1), lambda j: (0, 0)),
            ],
            out_specs=pl.BlockSpec((G, D, N), lambda j: (j, 0, 0)),
        ),
        compiler_params=pltpu.CompilerParams(
            dimension_semantics=("parallel",)),
    )(x, wr, wm, b0p, w1p, b1p, w2p, b2p)
    return out


# speedup vs baseline: 5.1323x; 1.0769x over previous
import functools

import jax
import jax.numpy as jnp
from jax.experimental import pallas as pl
from jax.experimental.pallas import tpu as pltpu


def _st_kernel(x_ref, w0_ref, b0_ref, w1_ref, b1_ref, w2_ref, b2_ref, o_ref,
               wr_s, wm_s, w1_s, w2_s, b0_s, b1_s, b2_s, *, kernel_size):
    G, S, N = x_ref.shape
    GN = G * N
    pad = (kernel_size - 1) // 2

    @pl.when(pl.program_id(0) == 0)
    def _prep():
        i = jax.lax.broadcasted_iota(jnp.int32, (S, S), 0).astype(jnp.float32)
        j = jax.lax.broadcasted_iota(jnp.int32, (S, S), 1).astype(jnp.float32)
        band = (jnp.abs(i - j) <= pad).astype(jnp.float32)
        left = jnp.where(j == 0.0, jnp.maximum(pad - i, 0.0), 0.0)
        right = jnp.where(j == float(S - 1),
                          jnp.maximum(i - float(S - 1 - pad), 0.0), 0.0)
        A = (band + left + right) * (1.0 / kernel_size)

        w0 = w0_ref[...]
        wm = jnp.dot(w0, A, preferred_element_type=jnp.float32)
        wm_s[...] = (0.5 * wm).astype(jnp.bfloat16)
        wr_s[...] = (0.5 * (w0 - wm)).astype(jnp.bfloat16)
        b0_s[...] = 0.5 * b0_ref[...]
        w1 = w1_ref[...]
        w1_s[...] = (0.25 * w1).astype(jnp.bfloat16)
        b1_s[...] = 0.5 * b1_ref[...] + 0.25 * jnp.sum(w1, axis=1,
                                                       keepdims=True)
        w2 = w2_ref[...]
        w2_s[...] = (0.25 * w2).astype(jnp.bfloat16)
        b2_s[...] = 0.5 * b2_ref[...] + 0.25 * jnp.sum(w2, axis=1,
                                                       keepdims=True)

    xc = jnp.concatenate([x_ref[g] for g in range(G)],
                         axis=1).astype(jnp.bfloat16)

    b0 = b0_s[...]
    ut = jnp.tanh(jnp.dot(wr_s[...], xc,
                          preferred_element_type=jnp.float32) + b0)
    ur = jnp.tanh(jnp.dot(wm_s[...], xc,
                          preferred_element_type=jnp.float32) + b0)
    v = jnp.concatenate([ut, ur], axis=1).astype(jnp.bfloat16)
    u = jnp.tanh(jnp.dot(w1_s[...], v,
                         preferred_element_type=jnp.float32) + b1_s[...])
    u = u.astype(jnp.bfloat16)
    h = jnp.tanh(jnp.dot(w2_s[...], u,
                         preferred_element_type=jnp.float32) + b2_s[...])
    for g in range(G):
        o_ref[g] = (1.0 + 0.5 * (h[:, g * N:(g + 1) * N] +
                                 h[:, GN + g * N:GN + (g + 1) * N])
                    ).astype(o_ref.dtype)


def kernel(x, w0, b0, w1, b1, w2, b2):
    B, S, N = x.shape
    D = w0.shape[0]
    kernel_size = 25

    G = 4 if B % 4 == 0 else 1
    body = functools.partial(_st_kernel, kernel_size=kernel_size)
    out = pl.pallas_call(
        body,
        out_shape=jax.ShapeDtypeStruct((B, D, N), x.dtype),
        grid_spec=pltpu.PrefetchScalarGridSpec(
            num_scalar_prefetch=0,
            grid=(B // G,),
            in_specs=[
                pl.BlockSpec((G, S, N), lambda j: (j, 0, 0)),
                pl.BlockSpec((D, S), lambda j: (0, 0)),
                pl.BlockSpec((D, 1), lambda j: (0, 0)),
                pl.BlockSpec((D, D), lambda j: (0, 0)),
                pl.BlockSpec((D, 1), lambda j: (0, 0)),
                pl.BlockSpec((D, D), lambda j: (0, 0)),
                pl.BlockSpec((D, 1), lambda j: (0, 0)),
            ],
            out_specs=pl.BlockSpec((G, D, N), lambda j: (j, 0, 0)),
            scratch_shapes=[
                pltpu.VMEM((D, S), jnp.bfloat16),
                pltpu.VMEM((D, S), jnp.bfloat16),
                pltpu.VMEM((D, D), jnp.bfloat16),
                pltpu.VMEM((D, D), jnp.bfloat16),
                pltpu.VMEM((D, 1), jnp.float32),
                pltpu.VMEM((D, 1), jnp.float32),
                pltpu.VMEM((D, 1), jnp.float32),
            ],
        ),
        compiler_params=pltpu.CompilerParams(
            dimension_semantics=("arbitrary",)),
    )(x, w0, b0.reshape(D, 1), w1, b1.reshape(D, 1), w2, b2.reshape(D, 1))
    return out
